# Initial kernel scaffold; baseline (speedup 1.0000x reference)
import jax
import jax.numpy as jnp
from jax.experimental import pallas as pl


def kernel(img):
    raise NotImplementedError("write your pallas kernel here")


# speedup vs baseline: 11.2529x; 11.2529x over previous
import functools

import jax
import jax.numpy as jnp
import numpy as np
from jax import lax
from jax.experimental import pallas as pl
from jax.experimental.pallas import tpu as pltpu
from jax.experimental.pallas import tpu_sc as plsc

_ANGLE = 30.0
_W = 512
_H = 512
_NPLANES = 384
_TILE = 128
_TPOS = 16
_WIN_R = 192
_WIN_C = 208
_NWORKERS = 32
_PLANES_PER_WORKER = _NPLANES // 2


def _host_bboxes():
    a = np.float32(_ANGLE * np.pi / 180.0)
    c = np.cos(a, dtype=np.float32)
    s = np.sin(a, dtype=np.float32)
    xm = (_W + 1) / 2.0
    ym = (_H + 1) / 2.0
    ii, jj = np.meshgrid(np.arange(_W, dtype=np.float64),
                         np.arange(_H, dtype=np.float64), indexing="ij")
    si = np.clip(np.round(c * (ii - xm) + s * (jj - ym)) + xm, 0, _W - 1)
    sj = np.clip(np.round(-s * (ii - xm) + c * (jj - ym)) + ym, 0, _H - 1)
    r0s, c0s = [], []
    for ti in range(4):
        for tj in range(4):
            bi, bj = ti * _TILE, tj * _TILE
            br = si[bi:bi + _TILE, bj:bj + _TILE]
            bc = sj[bi:bi + _TILE, bj:bj + _TILE]
            r0 = min(max(0, (int(br.min()) - 4) & ~7), _W - _WIN_R)
            c0 = min(max(0, (int(bc.min()) - 4) & ~15), _H - _WIN_C)
            assert min(int(br.max()) + 2, _W - 1) <= r0 + _WIN_R - 1
            assert min(int(bc.max()) + 2, _H - 1) <= c0 + _WIN_C - 1
            assert r0 <= max(int(br.min()) - 2, 0)
            assert c0 <= max(int(bc.min()) - 2, 0)
            r0s.append(r0)
            c0s.append(c0)
    return r0s, c0s

_R0S, _C0S = _host_bboxes()


def _rot_map():
    a = np.float32(_ANGLE * np.pi / 180.0)
    c = np.cos(a, dtype=np.float32)
    s = np.sin(a, dtype=np.float32)
    R = jnp.array([[c, s], [-s, c]], dtype=jnp.float32)
    xx, yy = jnp.meshgrid(jnp.arange(_W), jnp.arange(_H), indexing="ij")
    xx = xx.astype(jnp.float32)
    yy = yy.astype(jnp.float32)
    xm = (_W + 1) / 2.0
    ym = (_H + 1) / 2.0
    pts = jnp.stack([(xx - xm).reshape(-1), (yy - ym).reshape(-1)], axis=0)
    inds = jnp.round(R @ pts) + jnp.array([[xm], [ym]], dtype=jnp.float32)
    inds = jnp.where(inds < 0, 0.0, inds)
    i0 = jnp.where(inds[0] >= _W, _W - 1.0, inds[0])
    i1 = jnp.where(inds[1] >= _H, _H - 1.0, inds[1])
    si = i0.astype(jnp.int32).reshape(_W, _H)
    sj = i1.astype(jnp.int32).reshape(_W, _H)
    return si, sj


def _tile_rel_indices():
    si, sj = _rot_map()
    sit = si.reshape(4, _TILE, 4, _TILE).transpose(0, 2, 1, 3)
    sjt = sj.reshape(4, _TILE, 4, _TILE).transpose(0, 2, 1, 3)
    sit = sit.reshape(_TPOS, _TILE, _TILE)
    sjt = sjt.reshape(_TPOS, _TILE, _TILE)
    r0 = jnp.asarray(np.array(_R0S, dtype=np.int32))[:, None, None]
    c0 = jnp.asarray(np.array(_C0S, dtype=np.int32))[:, None, None]
    idx_r = (sit - r0).reshape(_TPOS * _TILE, _TILE)
    idx_c = (sjt - c0).reshape(_TPOS * _TILE, _TILE)
    return idx_r, idx_c


def _static_lookup(tp, values):
    out = jnp.int32(values[0])
    for k in range(1, len(values)):
        out = jnp.where(tp == k, jnp.int32(values[k]), out)
    return out


def _sc_rotate(img2d, idx_r, idx_c):
    mesh = plsc.VectorSubcoreMesh(core_axis_name="c", subcore_axis_name="s")

    @functools.partial(
        pl.kernel,
        out_type=jax.ShapeDtypeStruct((_NPLANES * _W, _H), jnp.float32),
        mesh=mesh,
        scratch_types=[
            pltpu.VMEM((_TILE, _TILE), jnp.int32),
            pltpu.VMEM((_TILE, _TILE), jnp.int32),
            pltpu.VMEM((_WIN_R, _WIN_C), jnp.float32),
            pltpu.VMEM((_TILE, _TILE), jnp.float32),
        ],
        compiler_params=pltpu.CompilerParams(
            use_tc_tiling_on_sc=False, needs_layout_passes=False),
    )
    def k(img_hbm, idxr_hbm, idxc_hbm, out_hbm, idxr_v, idxc_v, win_v, outb_v):
        cid = lax.axis_index("c")
        sid = lax.axis_index("s")
        wid = sid * 2 + cid
        tp = lax.rem(wid, _TPOS)
        pg = lax.div(wid, _TPOS)
        r0 = pl.multiple_of(_static_lookup(tp, _R0S), 8)
        c0 = pl.multiple_of(_static_lookup(tp, _C0S), 16)
        ti = lax.div(tp, 4)
        tj = lax.rem(tp, 4)
        i0 = ti * _TILE
        j0 = tj * _TILE

        pltpu.sync_copy(idxr_hbm.at[pl.ds(tp * _TILE, _TILE), :], idxr_v)
        pltpu.sync_copy(idxc_hbm.at[pl.ds(tp * _TILE, _TILE), :], idxc_v)

        def plane_body(n, carry):
            p = pg * _PLANES_PER_WORKER + n
            prow = pl.multiple_of(p * _W, _W)
            pltpu.sync_copy(
                img_hbm.at[pl.ds(prow + r0, _WIN_R), pl.ds(c0, _WIN_C)], win_v)

            def row_body(r, carry2):
                for u in range(_TILE // 16):
                    rv = idxr_v[r, pl.ds(u * 16, 16)]
                    cv = idxc_v[r, pl.ds(u * 16, 16)]
                    v = plsc.load_gather(win_v, [rv, cv])
                    outb_v[r, pl.ds(u * 16, 16)] = v
                return carry2

            lax.fori_loop(0, _TILE, row_body, 0)
            pltpu.sync_copy(
                outb_v, out_hbm.at[pl.ds(prow + i0, _TILE), pl.ds(j0, _TILE)])
            return carry

        lax.fori_loop(0, _PLANES_PER_WORKER, plane_body, 0)

    return k(img2d, idx_r, idx_c)


@jax.jit
def kernel(img):
    b, ch, w, h = img.shape
    idx_r, idx_c = _tile_rel_indices()
    img2d = img.reshape(b * ch * w, h)
    out2d = _sc_rotate(img2d, idx_r, idx_c)
    return out2d.reshape(b, ch, w, h)


# speedup vs baseline: 11.7619x; 1.0452x over previous
import functools

import jax
import jax.numpy as jnp
import numpy as np
from jax import lax
from jax.experimental import pallas as pl
from jax.experimental.pallas import tpu as pltpu
from jax.experimental.pallas import tpu_sc as plsc

_ANGLE = 30.0
_W = 512
_H = 512
_NPLANES = 384
_TILE = 128
_TPOS = 16
_WIN_R = 192
_WIN_C = 192
_PLANES_PER_WORKER = _NPLANES // 2


def _host_bboxes():
    a = np.float32(_ANGLE * np.pi / 180.0)
    c = np.cos(a, dtype=np.float32)
    s = np.sin(a, dtype=np.float32)
    xm = (_W + 1) / 2.0
    ym = (_H + 1) / 2.0
    ii, jj = np.meshgrid(np.arange(_W, dtype=np.float64),
                         np.arange(_H, dtype=np.float64), indexing="ij")
    si = np.clip(np.round(c * (ii - xm) + s * (jj - ym)) + xm, 0, _W - 1)
    sj = np.clip(np.round(-s * (ii - xm) + c * (jj - ym)) + ym, 0, _H - 1)
    r0s, c0s = [], []
    for ti in range(4):
        for tj in range(4):
            bi, bj = ti * _TILE, tj * _TILE
            br = si[bi:bi + _TILE, bj:bj + _TILE]
            bc = sj[bi:bi + _TILE, bj:bj + _TILE]
            r0 = min(max(0, (int(br.min()) - 4) & ~7), _W - _WIN_R)
            c0 = min(max(0, (int(bc.min()) - 4) & ~7), _H - _WIN_C)
            assert min(int(br.max()) + 2, _W - 1) <= r0 + _WIN_R - 1
            assert min(int(bc.max()) + 2, _H - 1) <= c0 + _WIN_C - 1
            assert r0 <= max(int(br.min()) - 2, 0)
            assert c0 <= max(int(bc.min()) - 2, 0)
            r0s.append(r0)
            c0s.append(c0)
    return r0s, c0s

_R0S, _C0S = _host_bboxes()


def _rot_map():
    a = np.float32(_ANGLE * np.pi / 180.0)
    c = np.cos(a, dtype=np.float32)
    s = np.sin(a, dtype=np.float32)
    R = jnp.array([[c, s], [-s, c]], dtype=jnp.float32)
    xx, yy = jnp.meshgrid(jnp.arange(_W), jnp.arange(_H), indexing="ij")
    xx = xx.astype(jnp.float32)
    yy = yy.astype(jnp.float32)
    xm = (_W + 1) / 2.0
    ym = (_H + 1) / 2.0
    pts = jnp.stack([(xx - xm).reshape(-1), (yy - ym).reshape(-1)], axis=0)
    inds = jnp.round(R @ pts) + jnp.array([[xm], [ym]], dtype=jnp.float32)
    inds = jnp.where(inds < 0, 0.0, inds)
    i0 = jnp.where(inds[0] >= _W, _W - 1.0, inds[0])
    i1 = jnp.where(inds[1] >= _H, _H - 1.0, inds[1])
    si = i0.astype(jnp.int32).reshape(_W, _H)
    sj = i1.astype(jnp.int32).reshape(_W, _H)
    return si, sj


def _tile_packed_indices():
    si, sj = _rot_map()
    sit = si.reshape(4, _TILE, 4, _TILE).transpose(0, 2, 1, 3)
    sjt = sj.reshape(4, _TILE, 4, _TILE).transpose(0, 2, 1, 3)
    sit = sit.reshape(_TPOS, _TILE, _TILE)
    sjt = sjt.reshape(_TPOS, _TILE, _TILE)
    r0 = jnp.asarray(np.array(_R0S, dtype=np.int32))[:, None, None]
    c0 = jnp.asarray(np.array(_C0S, dtype=np.int32))[:, None, None]
    packed = (sit - r0) * 256 + (sjt - c0)
    return packed.reshape(_TPOS * _TILE, _TILE)


def _static_lookup(tp, values):
    out = jnp.int32(values[0])
    for k in range(1, len(values)):
        out = jnp.where(tp == k, jnp.int32(values[k]), out)
    return out


def _sc_rotate(img2d, idx_pk):
    mesh = plsc.VectorSubcoreMesh(core_axis_name="c", subcore_axis_name="s")

    @functools.partial(
        pl.kernel,
        out_type=jax.ShapeDtypeStruct((_NPLANES * _W, _H), jnp.float32),
        mesh=mesh,
        scratch_types=[
            pltpu.VMEM((_TILE, _TILE), jnp.int32),
            pltpu.VMEM((_WIN_R, _WIN_C), jnp.float32),
            pltpu.VMEM((_WIN_R, _WIN_C), jnp.float32),
            pltpu.VMEM((_TILE, _TILE), jnp.float32),
            pltpu.VMEM((_TILE, _TILE), jnp.float32),
            pltpu.SemaphoreType.DMA,
            pltpu.SemaphoreType.DMA,
            pltpu.SemaphoreType.DMA,
            pltpu.SemaphoreType.DMA,
        ],
        compiler_params=pltpu.CompilerParams(
            use_tc_tiling_on_sc=False, needs_layout_passes=False),
    )
    def k(img_hbm, idx_hbm, out_hbm, idx_v, win0, win1, ob0, ob1,
          sw0, sw1, so0, so1):
        cid = lax.axis_index("c")
        sid = lax.axis_index("s")
        wid = sid * 2 + cid
        tp = lax.rem(wid, _TPOS)
        pg = lax.div(wid, _TPOS)
        r0 = pl.multiple_of(_static_lookup(tp, _R0S), 8)
        c0 = pl.multiple_of(_static_lookup(tp, _C0S), 8)
        ti = lax.div(tp, 4)
        tj = lax.rem(tp, 4)
        i0 = ti * _TILE
        j0 = tj * _TILE
        pbase = pg * _PLANES_PER_WORKER

        pltpu.sync_copy(idx_hbm.at[pl.ds(tp * _TILE, _TILE), :], idx_v)

        def win_src(n):
            prow = pl.multiple_of((pbase + n) * _W, _W)
            return img_hbm.at[pl.ds(prow + r0, _WIN_R), pl.ds(c0, _WIN_C)]

        def out_dst(n):
            prow = pl.multiple_of((pbase + n) * _W, _W)
            return out_hbm.at[pl.ds(prow + i0, _TILE), pl.ds(j0, _TILE)]

        def gather(win, ob):
            def row_body(r, carry2):
                for u in range(_TILE // 16):
                    pk = idx_v[r, pl.ds(u * 16, 16)]
                    rv = jax.lax.shift_right_logical(pk, 8)
                    cv = jnp.bitwise_and(pk, 255)
                    ob[r, pl.ds(u * 16, 16)] = plsc.load_gather(win, [rv, cv])
                return carry2

            lax.fori_loop(0, _TILE, row_body, 0)

        pltpu.async_copy(win_src(0), win0, sw0)
        pltpu.async_copy(win_src(1), win1, sw1)

        pltpu.make_async_copy(win_src(0), win0, sw0).wait()
        gather(win0, ob0)
        pltpu.async_copy(ob0, out_dst(0), so0)
        pltpu.async_copy(win_src(2), win0, sw0)

        pltpu.make_async_copy(win_src(1), win1, sw1).wait()
        gather(win1, ob1)
        pltpu.async_copy(ob1, out_dst(1), so1)
        pltpu.async_copy(win_src(3), win1, sw1)

        def pair_body(m, carry):
            p0 = 2 * m
            pltpu.make_async_copy(win_src(p0), win0, sw0).wait()
            pltpu.make_async_copy(ob0, out_dst(p0), so0).wait()
            gather(win0, ob0)
            pltpu.async_copy(ob0, out_dst(p0), so0)
            pltpu.async_copy(win_src(p0 + 2), win0, sw0)

            pltpu.make_async_copy(win_src(p0 + 1), win1, sw1).wait()
            pltpu.make_async_copy(ob1, out_dst(p0 + 1), so1).wait()
            gather(win1, ob1)
            pltpu.async_copy(ob1, out_dst(p0 + 1), so1)
            pltpu.async_copy(win_src(p0 + 3), win1, sw1)
            return carry

        lax.fori_loop(1, _PLANES_PER_WORKER // 2 - 1, pair_body, 0)

        pl_ = _PLANES_PER_WORKER - 2
        pltpu.make_async_copy(win_src(pl_), win0, sw0).wait()
        pltpu.make_async_copy(ob0, out_dst(pl_), so0).wait()
        gather(win0, ob0)
        pltpu.async_copy(ob0, out_dst(pl_), so0)

        pltpu.make_async_copy(win_src(pl_ + 1), win1, sw1).wait()
        pltpu.make_async_copy(ob1, out_dst(pl_ + 1), so1).wait()
        gather(win1, ob1)
        pltpu.async_copy(ob1, out_dst(pl_ + 1), so1)

        pltpu.make_async_copy(ob0, out_dst(pl_), so0).wait()
        pltpu.make_async_copy(ob1, out_dst(pl_ + 1), so1).wait()

    return k(img2d, idx_pk)


@jax.jit
def kernel(img):
    b, ch, w, h = img.shape
    idx_pk = _tile_packed_indices()
    img2d = img.reshape(b * ch * w, h)
    out2d = _sc_rotate(img2d, idx_pk)
    return out2d.reshape(b, ch, w, h)


# speedup vs baseline: 14.8163x; 1.2597x over previous
import functools

import jax
import jax.numpy as jnp
import numpy as np
from jax import lax
from jax.experimental import pallas as pl
from jax.experimental.pallas import tpu as pltpu
from jax.experimental.pallas import tpu_sc as plsc

_ANGLE = 30.0
_W = 512
_H = 512
_NPLANES = 384
_TILE = 128
_TPOS = 16
_WIN_R = 192
_WIN_C = 192
_PLANES_PER_WORKER = _NPLANES // 2


def _host_bboxes():
    a = np.float32(_ANGLE * np.pi / 180.0)
    c = np.cos(a, dtype=np.float32)
    s = np.sin(a, dtype=np.float32)
    xm = (_W + 1) / 2.0
    ym = (_H + 1) / 2.0
    ii, jj = np.meshgrid(np.arange(_W, dtype=np.float64),
                         np.arange(_H, dtype=np.float64), indexing="ij")
    si = np.clip(np.round(c * (ii - xm) + s * (jj - ym)) + xm, 0, _W - 1)
    sj = np.clip(np.round(-s * (ii - xm) + c * (jj - ym)) + ym, 0, _H - 1)
    r0s, c0s = [], []
    for ti in range(4):
        for tj in range(4):
            bi, bj = ti * _TILE, tj * _TILE
            br = si[bi:bi + _TILE, bj:bj + _TILE]
            bc = sj[bi:bi + _TILE, bj:bj + _TILE]
            r0 = min(max(0, (int(br.min()) - 4) & ~7), _W - _WIN_R)
            c0 = min(max(0, (int(bc.min()) - 4) & ~7), _H - _WIN_C)
            assert min(int(br.max()) + 2, _W - 1) <= r0 + _WIN_R - 1
            assert min(int(bc.max()) + 2, _H - 1) <= c0 + _WIN_C - 1
            assert r0 <= max(int(br.min()) - 2, 0)
            assert c0 <= max(int(bc.min()) - 2, 0)
            r0s.append(r0)
            c0s.append(c0)
    return r0s, c0s

_R0S, _C0S = _host_bboxes()


def _rot_map():
    a = np.float32(_ANGLE * np.pi / 180.0)
    c = np.cos(a, dtype=np.float32)
    s = np.sin(a, dtype=np.float32)
    R = jnp.array([[c, s], [-s, c]], dtype=jnp.float32)
    xx, yy = jnp.meshgrid(jnp.arange(_W), jnp.arange(_H), indexing="ij")
    xx = xx.astype(jnp.float32)
    yy = yy.astype(jnp.float32)
    xm = (_W + 1) / 2.0
    ym = (_H + 1) / 2.0
    pts = jnp.stack([(xx - xm).reshape(-1), (yy - ym).reshape(-1)], axis=0)
    inds = jnp.round(R @ pts) + jnp.array([[xm], [ym]], dtype=jnp.float32)
    inds = jnp.where(inds < 0, 0.0, inds)
    i0 = jnp.where(inds[0] >= _W, _W - 1.0, inds[0])
    i1 = jnp.where(inds[1] >= _H, _H - 1.0, inds[1])
    si = i0.astype(jnp.int32).reshape(_W, _H)
    sj = i1.astype(jnp.int32).reshape(_W, _H)
    return si, sj


def _tile_packed_indices():
    si, sj = _rot_map()
    sit = si.reshape(4, _TILE, 4, _TILE).transpose(0, 2, 1, 3)
    sjt = sj.reshape(4, _TILE, 4, _TILE).transpose(0, 2, 1, 3)
    sit = sit.reshape(_TPOS, _TILE, _TILE)
    sjt = sjt.reshape(_TPOS, _TILE, _TILE)
    r0 = jnp.asarray(np.array(_R0S, dtype=np.int32))[:, None, None]
    c0 = jnp.asarray(np.array(_C0S, dtype=np.int32))[:, None, None]
    packed = (sit - r0) * 256 + (sjt - c0)
    return packed.reshape(_TPOS * _TILE, _TILE)


def _static_lookup(tp, values):
    out = jnp.int32(values[0])
    for k in range(1, len(values)):
        out = jnp.where(tp == k, jnp.int32(values[k]), out)
    return out


def _sc_rotate(img2d, idx_pk):
    mesh = plsc.VectorSubcoreMesh(core_axis_name="c", subcore_axis_name="s")

    @functools.partial(
        pl.kernel,
        out_type=jax.ShapeDtypeStruct((_NPLANES * _W, _H), jnp.float32),
        mesh=mesh,
        scratch_types=[
            pltpu.VMEM((_TILE, _TILE), jnp.int32),
            pltpu.VMEM((_WIN_R, _WIN_C), jnp.float32),
            pltpu.VMEM((_WIN_R, _WIN_C), jnp.float32),
            pltpu.VMEM((_TILE, _TILE), jnp.float32),
            pltpu.VMEM((_TILE, _TILE), jnp.float32),
            pltpu.SemaphoreType.DMA,
            pltpu.SemaphoreType.DMA,
            pltpu.SemaphoreType.DMA,
            pltpu.SemaphoreType.DMA,
        ],
        compiler_params=pltpu.CompilerParams(
            use_tc_tiling_on_sc=False, needs_layout_passes=False),
    )
    def k(img_hbm, idx_hbm, out_hbm, idx_v, win0, win1, ob0, ob1,
          sw0, sw1, so0, so1):
        cid = lax.axis_index("c")
        sid = lax.axis_index("s")
        wid = sid * 2 + cid
        tp = lax.rem(wid, _TPOS)
        pg = lax.div(wid, _TPOS)
        r0 = pl.multiple_of(_static_lookup(tp, _R0S), 8)
        c0 = pl.multiple_of(_static_lookup(tp, _C0S), 8)
        ti = lax.div(tp, 4)
        tj = lax.rem(tp, 4)
        i0 = ti * _TILE
        j0 = tj * _TILE
        pbase = pg * _PLANES_PER_WORKER

        pltpu.sync_copy(idx_hbm.at[pl.ds(tp * _TILE, _TILE), :], idx_v)

        def win_src(n):
            prow = pl.multiple_of((pbase + n) * _W, _W)
            return img_hbm.at[pl.ds(prow + r0, _WIN_R), pl.ds(c0, _WIN_C)]

        def out_dst(n):
            prow = pl.multiple_of((pbase + n) * _W, _W)
            return out_hbm.at[pl.ds(prow + i0, _TILE), pl.ds(j0, _TILE)]

        def gather(win, ob):
            @plsc.parallel_loop(0, _TILE, step=1, unroll=2)
            def row_body(r):
                for u in range(_TILE // 16):
                    pk = idx_v[r, pl.ds(u * 16, 16)]
                    rv = jax.lax.shift_right_logical(pk, 8)
                    cv = jnp.bitwise_and(pk, 255)
                    ob[r, pl.ds(u * 16, 16)] = plsc.load_gather(win, [rv, cv])

        pltpu.async_copy(win_src(0), win0, sw0)
        pltpu.async_copy(win_src(1), win1, sw1)

        pltpu.make_async_copy(win_src(0), win0, sw0).wait()
        gather(win0, ob0)
        pltpu.async_copy(ob0, out_dst(0), so0)
        pltpu.async_copy(win_src(2), win0, sw0)

        pltpu.make_async_copy(win_src(1), win1, sw1).wait()
        gather(win1, ob1)
        pltpu.async_copy(ob1, out_dst(1), so1)
        pltpu.async_copy(win_src(3), win1, sw1)

        def pair_body(m, carry):
            p0 = 2 * m
            pltpu.make_async_copy(win_src(p0), win0, sw0).wait()
            pltpu.make_async_copy(ob0, out_dst(p0), so0).wait()
            gather(win0, ob0)
            pltpu.async_copy(ob0, out_dst(p0), so0)
            pltpu.async_copy(win_src(p0 + 2), win0, sw0)

            pltpu.make_async_copy(win_src(p0 + 1), win1, sw1).wait()
            pltpu.make_async_copy(ob1, out_dst(p0 + 1), so1).wait()
            gather(win1, ob1)
            pltpu.async_copy(ob1, out_dst(p0 + 1), so1)
            pltpu.async_copy(win_src(p0 + 3), win1, sw1)
            return carry

        lax.fori_loop(1, _PLANES_PER_WORKER // 2 - 1, pair_body, 0)

        pl_ = _PLANES_PER_WORKER - 2
        pltpu.make_async_copy(win_src(pl_), win0, sw0).wait()
        pltpu.make_async_copy(ob0, out_dst(pl_), so0).wait()
        gather(win0, ob0)
        pltpu.async_copy(ob0, out_dst(pl_), so0)

        pltpu.make_async_copy(win_src(pl_ + 1), win1, sw1).wait()
        pltpu.make_async_copy(ob1, out_dst(pl_ + 1), so1).wait()
        gather(win1, ob1)
        pltpu.async_copy(ob1, out_dst(pl_ + 1), so1)

        pltpu.make_async_copy(ob0, out_dst(pl_), so0).wait()
        pltpu.make_async_copy(ob1, out_dst(pl_ + 1), so1).wait()

    return k(img2d, idx_pk)


@jax.jit
def kernel(img):
    b, ch, w, h = img.shape
    idx_pk = _tile_packed_indices()
    img2d = img.reshape(b * ch * w, h)
    out2d = _sc_rotate(img2d, idx_pk)
    return out2d.reshape(b, ch, w, h)
